# baseline (speedup 1.0000x reference)
import functools
import jax
import jax.numpy as jnp
import numpy as np
from jax import lax
from jax.experimental import pallas as pl
from jax.experimental.pallas import tpu as pltpu
from jax.experimental.pallas import tpu_sc as plsc

_D = 512
_QD = 128
_NH = 8
_DH = _D // _NH
_K = 3
_STEPS = 5
_C = 100000
_V = 50257
_B = 1024

_CB = 2048
_TILES = _CB // 128
_NCB = (_C + _CB - 1) // _CB
_REM = _C - (_NCB - 1) * _CB
_LAST_TILES = (_REM + 127) // 128
_LAST_LANES = _REM - (_LAST_TILES - 1) * 128

_PB = 128

_SC_NC = 2
_SC_NS = 16
_NW = _SC_NC * _SC_NS
_NROWS = _B * _K
_RPW = _NROWS // _NW


def _gelu(x):
    return x * (lax.erf(x / np.sqrt(2).astype(np.float32)) + 1) / 2


def _ln(x, w, b, eps=1e-5):
    m = jnp.mean(x, axis=-1, keepdims=True)
    v = jnp.mean((x - m) * (x - m), axis=-1, keepdims=True)
    return (x - m) / jnp.sqrt(v + eps) * w + b


def _nrm(x, eps=1e-12):
    n = jnp.sqrt(jnp.sum(x * x, axis=-1, keepdims=True))
    return x / jnp.maximum(n, eps)


def _dott(a, w):
    return lax.dot_general(a, w, (((1,), (1,)), ((), ())),
                           preferred_element_type=jnp.float32)


def _qgen_compute(cur, ctx, wp1, bp1, wp2, bp2, lw, lb, wg, bg):
    h = _gelu(_dott(cur, wp1) + bp1)
    q = _ln(_dott(h, wp2) + bp2, lw, lb)
    if ctx is not None:
        cc = jnp.concatenate([cur, ctx], axis=1)
        q = q * jax.nn.sigmoid(_dott(cc, wg) + bg)
    q = _nrm(q)
    q = _nrm(q)
    return q


def _thought_compute(first_step, cur, docs, ctx, wq, bq, wk, bk, wv, bv,
                     wo, bo, wt1, bt1, ltw, ltb, wt2, bt2):
    nrows = cur.shape[0]
    qp = _dott(cur, wq) + bq
    ks = []
    vs = []
    for k in range(_K):
        dk = docs[:, k * _D:(k + 1) * _D]
        ks.append(_dott(dk, wk) + bk)
        vs.append(_dott(dk, wv) + bv)
    scale = np.float32(1.0 / np.sqrt(_DH))
    sks = []
    for k in range(_K):
        cols = []
        for h in range(_NH):
            sl = slice(h * _DH, (h + 1) * _DH)
            cols.append(jnp.sum(qp[:, sl] * ks[k][:, sl], axis=1,
                                keepdims=True) * scale)
        sks.append(jnp.concatenate(cols, axis=1))
    m = jnp.maximum(jnp.maximum(sks[0], sks[1]), sks[2])
    es = [jnp.exp(sk - m) for sk in sks]
    z = es[0] + es[1] + es[2]
    wsoft = [e / z for e in es]
    attn = None
    for k in range(_K):
        wf = jnp.concatenate(
            [lax.broadcast_in_dim(wsoft[k][:, h:h + 1], (nrows, _DH),
                                  (0, 1)) for h in range(_NH)], axis=1)
        contrib = wf * vs[k]
        attn = contrib if attn is None else attn + contrib
    ao = _dott(attn, wo) + bo
    prev = cur if first_step else ctx
    combined = jnp.concatenate([ao, prev], axis=1)
    h1 = _ln(_gelu(_dott(combined, wt1) + bt1), ltw, ltb)
    th = _dott(h1, wt2) + bt2
    nctx = th if first_step else 0.7 * th + 0.3 * ctx
    return th, nctx


def _stream_tiles(q, ck_ref, st, c, n_tiles, mask_tail):
    t1, t2, t3, i1, i2, i3 = st
    for t in range(n_tiles):
        ck_t = ck_ref[t * 128:(t + 1) * 128, :]
        s = _dott(q, ck_t)
        if mask_tail and t == n_tiles - 1:
            lane = lax.broadcasted_iota(jnp.int32, (_B, 128), 1)
            s = jnp.where(lane < _LAST_LANES, s, -jnp.inf)
        tid = lax.broadcast_in_dim(c * _TILES + t, (_B, 128), ())
        c1 = s > t1
        sp1 = jnp.minimum(s, t1)
        si1 = jnp.where(c1, i1, tid)
        t1 = jnp.maximum(s, t1)
        i1 = jnp.where(c1, tid, i1)
        c2 = sp1 > t2
        sp2 = jnp.minimum(sp1, t2)
        si2 = jnp.where(c2, i2, si1)
        t2 = jnp.maximum(sp1, t2)
        i2 = jnp.where(c2, si1, i2)
        c3 = sp2 > t3
        t3 = jnp.maximum(sp2, t3)
        i3 = jnp.where(c3, si2, i3)
    return t1, t2, t3, i1, i2, i3


def _collapse_top3(st, oi_ref):
    a1, a2, a3, b1, b2, b3 = st
    li = lax.broadcasted_iota(jnp.int32, (_B, 128), 1)
    outs = []
    for _j in range(_K):
        m = jnp.max(a1, axis=1)
        pos = jnp.min(jnp.where(a1 == m[:, None], li, 128), axis=1)
        sel = li == pos[:, None]
        gid = jnp.sum(jnp.where(sel, b1, 0), axis=1) * 128 + pos
        outs.append(gid[:, None])
        a1 = jnp.where(sel, a2, a1)
        b1 = jnp.where(sel, b2, b1)
        a2 = jnp.where(sel, a3, a2)
        b2 = jnp.where(sel, b3, b2)
        a3 = jnp.where(sel, -jnp.inf, a3)
    oi_ref[...] = jnp.concatenate(outs, axis=1)


def _load_st(t1r, t2r, t3r, i1r, i2r, i3r):
    return (t1r[...], t2r[...], t3r[...], i1r[...], i2r[...], i3r[...])


def _store_st(st, t1r, t2r, t3r, i1r, i2r, i3r):
    t1r[...], t2r[...], t3r[...] = st[0], st[1], st[2]
    i1r[...], i2r[...], i3r[...] = st[3], st[4], st[5]


def _init_st(t1r, t2r, t3r, i1r, i2r, i3r):
    neg = jnp.full((_B, 128), -jnp.inf, jnp.float32)
    zero = jnp.zeros((_B, 128), jnp.int32)
    t1r[...] = neg
    t2r[...] = neg
    t3r[...] = neg
    i1r[...] = zero
    i2r[...] = zero
    i3r[...] = zero


_SIMTOP_SCRATCH = [
    pltpu.VMEM((_B, 128), jnp.float32),
    pltpu.VMEM((_B, 128), jnp.float32),
    pltpu.VMEM((_B, 128), jnp.float32),
    pltpu.VMEM((_B, 128), jnp.int32),
    pltpu.VMEM((_B, 128), jnp.int32),
    pltpu.VMEM((_B, 128), jnp.int32),
    pltpu.VMEM((_B, _QD), jnp.float32),
]


def _simtop_main(c, ck_ref, oi_ref, scr):
    (t1r, t2r, t3r, i1r, i2r, i3r, qs_ref) = scr
    q = qs_ref[...]

    @pl.when(c < _NCB - 1)
    def _():
        st = _load_st(t1r, t2r, t3r, i1r, i2r, i3r)
        st = _stream_tiles(q, ck_ref, st, c, _TILES, False)
        _store_st(st, t1r, t2r, t3r, i1r, i2r, i3r)

    @pl.when(c == _NCB - 1)
    def _():
        st = _load_st(t1r, t2r, t3r, i1r, i2r, i3r)
        st = _stream_tiles(q, ck_ref, st, c, _LAST_TILES, True)
        _collapse_top3(st, oi_ref)


def _stepA_body(cur_ref, wp1_ref, bp1_ref, wp2_ref, bp2_ref, lw_ref,
                lb_ref, ck_ref, oi_ref, *scr):
    c = pl.program_id(0)

    @pl.when(c == 0)
    def _():
        for i in range(_B // _PB):
            sl = slice(i * _PB, (i + 1) * _PB)
            scr[6][sl, :] = _qgen_compute(
                cur_ref[sl, :], None, wp1_ref[...], bp1_ref[...],
                wp2_ref[...], bp2_ref[...], lw_ref[...], lb_ref[...],
                None, None)
        _init_st(*scr[:6])

    _simtop_main(c, ck_ref, oi_ref, scr)


def _stepA_call(cur, corpus_norm, wp1, bp1, wp2, bp2, lqw, lqb):
    cst = lambda c: (0, 0)
    return pl.pallas_call(
        _stepA_body,
        grid=(_NCB,),
        in_specs=[
            pl.BlockSpec((_B, _D), cst),
            pl.BlockSpec((_D, _D), cst),
            pl.BlockSpec((1, _D), cst),
            pl.BlockSpec((_QD, _D), cst),
            pl.BlockSpec((1, _QD), cst),
            pl.BlockSpec((1, _QD), cst),
            pl.BlockSpec((1, _QD), cst),
            pl.BlockSpec((_CB, _QD), lambda c: (c, 0)),
        ],
        out_specs=pl.BlockSpec((_B, _K), cst),
        out_shape=jax.ShapeDtypeStruct((_B, _K), jnp.int32),
        scratch_shapes=_SIMTOP_SCRATCH,
    )(cur, wp1, bp1.reshape(1, _D), wp2, bp2.reshape(1, _QD),
      lqw.reshape(1, _QD), lqb.reshape(1, _QD), corpus_norm)


def _stepB_body(first_thought, cur_ref, ctx_ref, docs_ref, wq_ref, bq_ref,
                wk_ref, bk_ref, wv_ref, bv_ref, wo_ref, bo_ref, wt1_ref,
                bt1_ref, ltw_ref, ltb_ref, wt2_ref, bt2_ref, wp1_ref,
                bp1_ref, wp2_ref, bp2_ref, lw_ref, lb_ref, wg_ref, bg_ref,
                ck_ref, oi_ref, ocur_ref, octx_ref, *scr):
    c = pl.program_id(0)

    @pl.when(c == 0)
    def _():
        for i in range(_B // _PB):
            sl = slice(i * _PB, (i + 1) * _PB)
            th, nctx = _thought_compute(
                first_thought, cur_ref[sl, :], docs_ref[sl, :],
                ctx_ref[sl, :], wq_ref[...], bq_ref[...], wk_ref[...],
                bk_ref[...], wv_ref[...], bv_ref[...], wo_ref[...],
                bo_ref[...], wt1_ref[...], bt1_ref[...], ltw_ref[...],
                ltb_ref[...], wt2_ref[...], bt2_ref[...])
            ocur_ref[sl, :] = th
            octx_ref[sl, :] = nctx
            scr[6][sl, :] = _qgen_compute(
                th, nctx, wp1_ref[...], bp1_ref[...], wp2_ref[...],
                bp2_ref[...], lw_ref[...], lb_ref[...], wg_ref[...],
                bg_ref[...])
        _init_st(*scr[:6])

    _simtop_main(c, ck_ref, oi_ref, scr)


def _stepB_call(first_thought, cur, ctx, docs, corpus_norm, wq, bq, wk,
                bk, wv, bv, wo, bo, wt1, bt1, ltw, ltb, wt2, bt2, wp1,
                bp1, wp2, bp2, lqw, lqb, wg, bg):
    body = functools.partial(_stepB_body, first_thought)
    cst = lambda c: (0, 0)
    row = pl.BlockSpec((_B, _D), cst)
    wsq = pl.BlockSpec((_D, _D), cst)
    bsp = pl.BlockSpec((1, _D), cst)
    bq_ = pl.BlockSpec((1, _QD), cst)
    return pl.pallas_call(
        body,
        grid=(_NCB,),
        in_specs=[
            row, row, pl.BlockSpec((_B, _K * _D), cst),
            wsq, bsp, wsq, bsp, wsq, bsp, wsq, bsp,
            pl.BlockSpec((_D, 2 * _D), cst), bsp, bsp, bsp, wsq, bsp,
            wsq, bsp, pl.BlockSpec((_QD, _D), cst), bq_, bq_, bq_,
            pl.BlockSpec((_QD, 2 * _D), cst), bq_,
            pl.BlockSpec((_CB, _QD), lambda c: (c, 0)),
        ],
        out_specs=(pl.BlockSpec((_B, _K), cst), row, row),
        out_shape=(
            jax.ShapeDtypeStruct((_B, _K), jnp.int32),
            jax.ShapeDtypeStruct((_B, _D), jnp.float32),
            jax.ShapeDtypeStruct((_B, _D), jnp.float32),
        ),
        scratch_shapes=_SIMTOP_SCRATCH,
    )(cur, ctx, docs, wq, bq.reshape(1, _D), wk, bk.reshape(1, _D),
      wv, bv.reshape(1, _D), wo, bo.reshape(1, _D), wt1,
      bt1.reshape(1, _D), ltw.reshape(1, _D), ltb.reshape(1, _D), wt2,
      bt2.reshape(1, _D), wp1, bp1.reshape(1, _D), wp2,
      bp2.reshape(1, _QD), lqw.reshape(1, _QD), lqb.reshape(1, _QD),
      wg, bg.reshape(1, _QD), corpus_norm)


_VB = 2048
_NVB = (_V + _VB - 1) // _VB


def _final_body(cur_ref, ctx_ref, docs_ref, wq_ref, bq_ref, wk_ref,
                bk_ref, wv_ref, bv_ref, wo_ref, bo_ref, wt1_ref, bt1_ref,
                ltw_ref, ltb_ref, wt2_ref, bt2_ref, wa1_ref, ba1_ref,
                w2_ref, b2_ref, o_ref, h_ref):
    c = pl.program_id(0)

    @pl.when(c == 0)
    def _():
        for i in range(_B // _PB):
            sl = slice(i * _PB, (i + 1) * _PB)
            th, _unused = _thought_compute(
                False, cur_ref[sl, :], docs_ref[sl, :], ctx_ref[sl, :],
                wq_ref[...], bq_ref[...], wk_ref[...], bk_ref[...],
                wv_ref[...], bv_ref[...], wo_ref[...], bo_ref[...],
                wt1_ref[...], bt1_ref[...], ltw_ref[...], ltb_ref[...],
                wt2_ref[...], bt2_ref[...])
            h_ref[sl, :] = _gelu(_dott(th, wa1_ref[...]) + ba1_ref[...])

    o_ref[...] = _dott(h_ref[...], w2_ref[...]) + b2_ref[...]


def _final_call(cur, ctx, docs, wq, bq, wk, bk, wv, bv, wo, bo, wt1, bt1,
                ltw, ltb, wt2, bt2, wa1, ba1, wa2, ba2):
    cst = lambda c: (0, 0)
    row = pl.BlockSpec((_B, _D), cst)
    wsq = pl.BlockSpec((_D, _D), cst)
    bsp = pl.BlockSpec((1, _D), cst)
    return pl.pallas_call(
        _final_body,
        grid=(_NVB,),
        in_specs=[
            row, row, pl.BlockSpec((_B, _K * _D), cst),
            wsq, bsp, wsq, bsp, wsq, bsp, wsq, bsp,
            pl.BlockSpec((_D, 2 * _D), cst), bsp, bsp, bsp, wsq, bsp,
            wsq, bsp,
            pl.BlockSpec((_VB, _D), lambda c: (c, 0)),
            pl.BlockSpec((1, _VB), lambda c: (0, c)),
        ],
        out_specs=pl.BlockSpec((_B, _VB), lambda c: (0, c)),
        out_shape=jax.ShapeDtypeStruct((_B, _V), jnp.float32),
        scratch_shapes=[pltpu.VMEM((_B, _D), jnp.float32)],
    )(cur, ctx, docs, wq, bq.reshape(1, _D), wk, bk.reshape(1, _D),
      wv, bv.reshape(1, _D), wo, bo.reshape(1, _D), wt1,
      bt1.reshape(1, _D), ltw.reshape(1, _D), ltb.reshape(1, _D), wt2,
      bt2.reshape(1, _D), wa1, ba1.reshape(1, _D), wa2,
      ba2.reshape(1, _V))


def _input_body(x_ref, w_ref, b_ref, lw_ref, lb_ref, o_ref):
    h = _dott(x_ref[...], w_ref[...]) + b_ref[...]
    o_ref[...] = _ln(_gelu(h), lw_ref[...], lb_ref[...])


def _input_call(x, w, b, lw, lb):
    return pl.pallas_call(
        _input_body,
        out_shape=jax.ShapeDtypeStruct((_B, _D), jnp.float32),
    )(x, w, b.reshape(1, _D), lw.reshape(1, _D), lb.reshape(1, _D))


def _cnorm_body(x_ref, o_ref):
    o_ref[...] = _nrm(x_ref[...])


def _cnorm_call(ck):
    blk = 2000
    return pl.pallas_call(
        _cnorm_body,
        grid=(_C // blk,),
        in_specs=[pl.BlockSpec((blk, _QD), lambda c: (c, 0))],
        out_specs=pl.BlockSpec((blk, _QD), lambda c: (c, 0)),
        out_shape=jax.ShapeDtypeStruct((_C, _QD), jnp.float32),
    )(ck)


@functools.cache
def _make_sc_gather():
    mesh = plsc.VectorSubcoreMesh(core_axis_name="c", subcore_axis_name="s")

    @functools.partial(
        pl.kernel, mesh=mesh,
        out_type=jax.ShapeDtypeStruct((_NROWS, _D), jnp.float32),
        scratch_types=[
            pltpu.VMEM((_RPW,), jnp.int32),
            pltpu.VMEM((_RPW, _D), jnp.float32),
            pltpu.SemaphoreType.DMA,
        ],
    )
    def sc_gather(table_hbm, idx_hbm, out_hbm, idx_v, rows_v, sem):
        wid = lax.axis_index("s") * _SC_NC + lax.axis_index("c")
        base = wid * _RPW
        pltpu.sync_copy(idx_hbm.at[pl.ds(base, _RPW)], idx_v)
        pltpu.async_copy(table_hbm.at[idx_v], rows_v, sem).wait()
        pltpu.sync_copy(rows_v, out_hbm.at[pl.ds(base, _RPW)])

    return sc_gather


def _sc_gather(table, idx):
    return _make_sc_gather()(table, idx)


def kernel(input_repr, W_in1, b_in1, ln_in_w, ln_in_b, Wp1, bp1, Wp2, bp2,
           lnq_w, lnq_b, Wg, bg, attn_in_w, attn_in_b, attn_out_w,
           attn_out_b, Wt1, bt1, lnt_w, lnt_b, Wt2, bt2, Wc1, bc1, Wc2,
           bc2, Wa1, ba1, Wa2, ba2, corpus_keys, corpus_values):
    Wq = attn_in_w[:_D]
    Wk = attn_in_w[_D:2 * _D]
    Wv = attn_in_w[2 * _D:]
    bq = attn_in_b[:_D]
    bk = attn_in_b[_D:2 * _D]
    bv = attn_in_b[2 * _D:]

    current = _input_call(input_repr, W_in1, b_in1, ln_in_w, ln_in_b)
    corpus_norm = _cnorm_call(corpus_keys)

    tw = (Wq, bq, Wk, bk, Wv, bv, attn_out_w, attn_out_b, Wt1, bt1,
          lnt_w, lnt_b, Wt2, bt2)
    qw = (Wp1, bp1, Wp2, bp2, lnq_w, lnq_b, Wg, bg)

    idx = _stepA_call(current, corpus_norm, Wp1, bp1, Wp2, bp2,
                      lnq_w, lnq_b)
    docs = _sc_gather(corpus_values, idx.reshape(_NROWS))
    docs = docs.reshape(_B, _K * _D)
    ctx = current
    for step in range(1, _STEPS):
        idx, current, ctx = _stepB_call(step == 1, current, ctx, docs,
                                        corpus_norm, *tw, *qw)
        docs = _sc_gather(corpus_values, idx.reshape(_NROWS))
        docs = docs.reshape(_B, _K * _D)
    return _final_call(current, ctx, docs, *tw, Wa1, ba1, Wa2, ba2)


# speedup vs baseline: 1.0472x; 1.0472x over previous
import functools
import jax
import jax.numpy as jnp
import numpy as np
from jax import lax
from jax.experimental import pallas as pl
from jax.experimental.pallas import tpu as pltpu
from jax.experimental.pallas import tpu_sc as plsc

_D = 512
_QD = 128
_NH = 8
_DH = _D // _NH
_K = 3
_STEPS = 5
_C = 100000
_V = 50257
_B = 1024

_CB = 4096
_TILES = _CB // 128
_NCB = (_C + _CB - 1) // _CB
_REM = _C - (_NCB - 1) * _CB
_LAST_TILES = (_REM + 127) // 128
_LAST_LANES = _REM - (_LAST_TILES - 1) * 128

_PB = 256

_SC_NC = 2
_SC_NS = 16
_NW = _SC_NC * _SC_NS
_NROWS = _B * _K
_RPW = _NROWS // _NW


def _gelu(x):
    return x * (lax.erf(x / np.sqrt(2).astype(np.float32)) + 1) / 2


def _ln(x, w, b, eps=1e-5):
    m = jnp.mean(x, axis=-1, keepdims=True)
    v = jnp.mean((x - m) * (x - m), axis=-1, keepdims=True)
    return (x - m) / jnp.sqrt(v + eps) * w + b


def _nrm(x, eps=1e-12):
    n = jnp.sqrt(jnp.sum(x * x, axis=-1, keepdims=True))
    return x / jnp.maximum(n, eps)


def _dott(a, w):
    return lax.dot_general(a, w, (((1,), (1,)), ((), ())),
                           preferred_element_type=jnp.float32)


def _qgen_compute(cur, ctx, wp1, bp1, wp2, bp2, lw, lb, wg, bg):
    h = _gelu(_dott(cur, wp1) + bp1)
    q = _ln(_dott(h, wp2) + bp2, lw, lb)
    if ctx is not None:
        cc = jnp.concatenate([cur, ctx], axis=1)
        q = q * jax.nn.sigmoid(_dott(cc, wg) + bg)
    q = _nrm(q)
    q = _nrm(q)
    return q


def _thought_compute(first_step, cur, docs, ctx, wq, bq, wk, bk, wv, bv,
                     wo, bo, wt1, bt1, ltw, ltb, wt2, bt2):
    nrows = cur.shape[0]
    qp = _dott(cur, wq) + bq
    ks = []
    vs = []
    for k in range(_K):
        dk = docs[:, k * _D:(k + 1) * _D]
        ks.append(_dott(dk, wk) + bk)
        vs.append(_dott(dk, wv) + bv)
    scale = np.float32(1.0 / np.sqrt(_DH))
    sks = []
    for k in range(_K):
        cols = []
        for h in range(_NH):
            sl = slice(h * _DH, (h + 1) * _DH)
            cols.append(jnp.sum(qp[:, sl] * ks[k][:, sl], axis=1,
                                keepdims=True) * scale)
        sks.append(jnp.concatenate(cols, axis=1))
    m = jnp.maximum(jnp.maximum(sks[0], sks[1]), sks[2])
    es = [jnp.exp(sk - m) for sk in sks]
    z = es[0] + es[1] + es[2]
    wsoft = [e / z for e in es]
    attn = None
    for k in range(_K):
        wf = jnp.concatenate(
            [lax.broadcast_in_dim(wsoft[k][:, h:h + 1], (nrows, _DH),
                                  (0, 1)) for h in range(_NH)], axis=1)
        contrib = wf * vs[k]
        attn = contrib if attn is None else attn + contrib
    ao = _dott(attn, wo) + bo
    prev = cur if first_step else ctx
    combined = jnp.concatenate([ao, prev], axis=1)
    h1 = _ln(_gelu(_dott(combined, wt1) + bt1), ltw, ltb)
    th = _dott(h1, wt2) + bt2
    nctx = th if first_step else 0.7 * th + 0.3 * ctx
    return th, nctx


def _stream_tiles(q, ck_ref, st, c, n_tiles, mask_tail):
    t1, t2, t3, i1, i2, i3 = st
    for t in range(n_tiles):
        ck_t = ck_ref[t * 128:(t + 1) * 128, :]
        s = _dott(q, ck_t)
        if mask_tail and t == n_tiles - 1:
            lane = lax.broadcasted_iota(jnp.int32, (_B, 128), 1)
            s = jnp.where(lane < _LAST_LANES, s, -jnp.inf)
        tid = lax.broadcast_in_dim(c * _TILES + t, (_B, 128), ())
        c1 = s > t1
        sp1 = jnp.minimum(s, t1)
        si1 = jnp.where(c1, i1, tid)
        t1 = jnp.maximum(s, t1)
        i1 = jnp.where(c1, tid, i1)
        c2 = sp1 > t2
        sp2 = jnp.minimum(sp1, t2)
        si2 = jnp.where(c2, i2, si1)
        t2 = jnp.maximum(sp1, t2)
        i2 = jnp.where(c2, si1, i2)
        c3 = sp2 > t3
        t3 = jnp.maximum(sp2, t3)
        i3 = jnp.where(c3, si2, i3)
    return t1, t2, t3, i1, i2, i3


def _collapse_top3(st, oi_ref):
    a1, a2, a3, b1, b2, b3 = st
    li = lax.broadcasted_iota(jnp.int32, (_B, 128), 1)
    outs = []
    for _j in range(_K):
        m = jnp.max(a1, axis=1)
        pos = jnp.min(jnp.where(a1 == m[:, None], li, 128), axis=1)
        sel = li == pos[:, None]
        gid = jnp.sum(jnp.where(sel, b1, 0), axis=1) * 128 + pos
        outs.append(gid[:, None])
        a1 = jnp.where(sel, a2, a1)
        b1 = jnp.where(sel, b2, b1)
        a2 = jnp.where(sel, a3, a2)
        b2 = jnp.where(sel, b3, b2)
        a3 = jnp.where(sel, -jnp.inf, a3)
    oi_ref[...] = jnp.concatenate(outs, axis=1)


def _load_st(t1r, t2r, t3r, i1r, i2r, i3r):
    return (t1r[...], t2r[...], t3r[...], i1r[...], i2r[...], i3r[...])


def _store_st(st, t1r, t2r, t3r, i1r, i2r, i3r):
    t1r[...], t2r[...], t3r[...] = st[0], st[1], st[2]
    i1r[...], i2r[...], i3r[...] = st[3], st[4], st[5]


def _init_st(t1r, t2r, t3r, i1r, i2r, i3r):
    neg = jnp.full((_B, 128), -jnp.inf, jnp.float32)
    zero = jnp.zeros((_B, 128), jnp.int32)
    t1r[...] = neg
    t2r[...] = neg
    t3r[...] = neg
    i1r[...] = zero
    i2r[...] = zero
    i3r[...] = zero


_SIMTOP_SCRATCH = [
    pltpu.VMEM((_B, 128), jnp.float32),
    pltpu.VMEM((_B, 128), jnp.float32),
    pltpu.VMEM((_B, 128), jnp.float32),
    pltpu.VMEM((_B, 128), jnp.int32),
    pltpu.VMEM((_B, 128), jnp.int32),
    pltpu.VMEM((_B, 128), jnp.int32),
    pltpu.VMEM((_B, _QD), jnp.float32),
]


def _simtop_main(c, ck_ref, oi_ref, scr):
    (t1r, t2r, t3r, i1r, i2r, i3r, qs_ref) = scr
    q = qs_ref[...]

    @pl.when(c < _NCB - 1)
    def _():
        st = _load_st(t1r, t2r, t3r, i1r, i2r, i3r)
        st = _stream_tiles(q, ck_ref, st, c, _TILES, False)
        _store_st(st, t1r, t2r, t3r, i1r, i2r, i3r)

    @pl.when(c == _NCB - 1)
    def _():
        st = _load_st(t1r, t2r, t3r, i1r, i2r, i3r)
        st = _stream_tiles(q, ck_ref, st, c, _LAST_TILES, True)
        _collapse_top3(st, oi_ref)


def _stepA_body(cur_ref, wp1_ref, bp1_ref, wp2_ref, bp2_ref, lw_ref,
                lb_ref, ck_ref, oi_ref, *scr):
    c = pl.program_id(0)

    @pl.when(c == 0)
    def _():
        for i in range(_B // _PB):
            sl = slice(i * _PB, (i + 1) * _PB)
            scr[6][sl, :] = _qgen_compute(
                cur_ref[sl, :], None, wp1_ref[...], bp1_ref[...],
                wp2_ref[...], bp2_ref[...], lw_ref[...], lb_ref[...],
                None, None)
        _init_st(*scr[:6])

    _simtop_main(c, ck_ref, oi_ref, scr)


def _stepA_call(cur, corpus_norm, wp1, bp1, wp2, bp2, lqw, lqb):
    cst = lambda c: (0, 0)
    return pl.pallas_call(
        _stepA_body,
        grid=(_NCB,),
        in_specs=[
            pl.BlockSpec((_B, _D), cst),
            pl.BlockSpec((_D, _D), cst),
            pl.BlockSpec((1, _D), cst),
            pl.BlockSpec((_QD, _D), cst),
            pl.BlockSpec((1, _QD), cst),
            pl.BlockSpec((1, _QD), cst),
            pl.BlockSpec((1, _QD), cst),
            pl.BlockSpec((_CB, _QD), lambda c: (c, 0)),
        ],
        out_specs=pl.BlockSpec((_B, _K), cst),
        out_shape=jax.ShapeDtypeStruct((_B, _K), jnp.int32),
        scratch_shapes=_SIMTOP_SCRATCH,
    )(cur, wp1, bp1.reshape(1, _D), wp2, bp2.reshape(1, _QD),
      lqw.reshape(1, _QD), lqb.reshape(1, _QD), corpus_norm)


def _stepB_body(first_thought, cur_ref, ctx_ref, docs_ref, wq_ref, bq_ref,
                wk_ref, bk_ref, wv_ref, bv_ref, wo_ref, bo_ref, wt1_ref,
                bt1_ref, ltw_ref, ltb_ref, wt2_ref, bt2_ref, wp1_ref,
                bp1_ref, wp2_ref, bp2_ref, lw_ref, lb_ref, wg_ref, bg_ref,
                ck_ref, oi_ref, ocur_ref, octx_ref, *scr):
    c = pl.program_id(0)

    @pl.when(c == 0)
    def _():
        for i in range(_B // _PB):
            sl = slice(i * _PB, (i + 1) * _PB)
            th, nctx = _thought_compute(
                first_thought, cur_ref[sl, :], docs_ref[sl, :],
                ctx_ref[sl, :], wq_ref[...], bq_ref[...], wk_ref[...],
                bk_ref[...], wv_ref[...], bv_ref[...], wo_ref[...],
                bo_ref[...], wt1_ref[...], bt1_ref[...], ltw_ref[...],
                ltb_ref[...], wt2_ref[...], bt2_ref[...])
            ocur_ref[sl, :] = th
            octx_ref[sl, :] = nctx
            scr[6][sl, :] = _qgen_compute(
                th, nctx, wp1_ref[...], bp1_ref[...], wp2_ref[...],
                bp2_ref[...], lw_ref[...], lb_ref[...], wg_ref[...],
                bg_ref[...])
        _init_st(*scr[:6])

    _simtop_main(c, ck_ref, oi_ref, scr)


def _stepB_call(first_thought, cur, ctx, docs, corpus_norm, wq, bq, wk,
                bk, wv, bv, wo, bo, wt1, bt1, ltw, ltb, wt2, bt2, wp1,
                bp1, wp2, bp2, lqw, lqb, wg, bg):
    body = functools.partial(_stepB_body, first_thought)
    cst = lambda c: (0, 0)
    row = pl.BlockSpec((_B, _D), cst)
    wsq = pl.BlockSpec((_D, _D), cst)
    bsp = pl.BlockSpec((1, _D), cst)
    bq_ = pl.BlockSpec((1, _QD), cst)
    return pl.pallas_call(
        body,
        grid=(_NCB,),
        in_specs=[
            row, row, pl.BlockSpec((_B, _K * _D), cst),
            wsq, bsp, wsq, bsp, wsq, bsp, wsq, bsp,
            pl.BlockSpec((_D, 2 * _D), cst), bsp, bsp, bsp, wsq, bsp,
            wsq, bsp, pl.BlockSpec((_QD, _D), cst), bq_, bq_, bq_,
            pl.BlockSpec((_QD, 2 * _D), cst), bq_,
            pl.BlockSpec((_CB, _QD), lambda c: (c, 0)),
        ],
        out_specs=(pl.BlockSpec((_B, _K), cst), row, row),
        out_shape=(
            jax.ShapeDtypeStruct((_B, _K), jnp.int32),
            jax.ShapeDtypeStruct((_B, _D), jnp.float32),
            jax.ShapeDtypeStruct((_B, _D), jnp.float32),
        ),
        scratch_shapes=_SIMTOP_SCRATCH,
    )(cur, ctx, docs, wq, bq.reshape(1, _D), wk, bk.reshape(1, _D),
      wv, bv.reshape(1, _D), wo, bo.reshape(1, _D), wt1,
      bt1.reshape(1, _D), ltw.reshape(1, _D), ltb.reshape(1, _D), wt2,
      bt2.reshape(1, _D), wp1, bp1.reshape(1, _D), wp2,
      bp2.reshape(1, _QD), lqw.reshape(1, _QD), lqb.reshape(1, _QD),
      wg, bg.reshape(1, _QD), corpus_norm)


_VB = 2048
_NVB = (_V + _VB - 1) // _VB


def _final_body(cur_ref, ctx_ref, docs_ref, wq_ref, bq_ref, wk_ref,
                bk_ref, wv_ref, bv_ref, wo_ref, bo_ref, wt1_ref, bt1_ref,
                ltw_ref, ltb_ref, wt2_ref, bt2_ref, wa1_ref, ba1_ref,
                w2_ref, b2_ref, o_ref, h_ref):
    c = pl.program_id(0)

    @pl.when(c == 0)
    def _():
        for i in range(_B // _PB):
            sl = slice(i * _PB, (i + 1) * _PB)
            th, _unused = _thought_compute(
                False, cur_ref[sl, :], docs_ref[sl, :], ctx_ref[sl, :],
                wq_ref[...], bq_ref[...], wk_ref[...], bk_ref[...],
                wv_ref[...], bv_ref[...], wo_ref[...], bo_ref[...],
                wt1_ref[...], bt1_ref[...], ltw_ref[...], ltb_ref[...],
                wt2_ref[...], bt2_ref[...])
            h_ref[sl, :] = _gelu(_dott(th, wa1_ref[...]) + ba1_ref[...])

    o_ref[...] = _dott(h_ref[...], w2_ref[...]) + b2_ref[...]


def _final_call(cur, ctx, docs, wq, bq, wk, bk, wv, bv, wo, bo, wt1, bt1,
                ltw, ltb, wt2, bt2, wa1, ba1, wa2, ba2):
    cst = lambda c: (0, 0)
    row = pl.BlockSpec((_B, _D), cst)
    wsq = pl.BlockSpec((_D, _D), cst)
    bsp = pl.BlockSpec((1, _D), cst)
    return pl.pallas_call(
        _final_body,
        grid=(_NVB,),
        in_specs=[
            row, row, pl.BlockSpec((_B, _K * _D), cst),
            wsq, bsp, wsq, bsp, wsq, bsp, wsq, bsp,
            pl.BlockSpec((_D, 2 * _D), cst), bsp, bsp, bsp, wsq, bsp,
            wsq, bsp,
            pl.BlockSpec((_VB, _D), lambda c: (c, 0)),
            pl.BlockSpec((1, _VB), lambda c: (0, c)),
        ],
        out_specs=pl.BlockSpec((_B, _VB), lambda c: (0, c)),
        out_shape=jax.ShapeDtypeStruct((_B, _V), jnp.float32),
        scratch_shapes=[pltpu.VMEM((_B, _D), jnp.float32)],
    )(cur, ctx, docs, wq, bq.reshape(1, _D), wk, bk.reshape(1, _D),
      wv, bv.reshape(1, _D), wo, bo.reshape(1, _D), wt1,
      bt1.reshape(1, _D), ltw.reshape(1, _D), ltb.reshape(1, _D), wt2,
      bt2.reshape(1, _D), wa1, ba1.reshape(1, _D), wa2,
      ba2.reshape(1, _V))


def _input_body(x_ref, w_ref, b_ref, lw_ref, lb_ref, o_ref):
    h = _dott(x_ref[...], w_ref[...]) + b_ref[...]
    o_ref[...] = _ln(_gelu(h), lw_ref[...], lb_ref[...])


def _input_call(x, w, b, lw, lb):
    return pl.pallas_call(
        _input_body,
        out_shape=jax.ShapeDtypeStruct((_B, _D), jnp.float32),
    )(x, w, b.reshape(1, _D), lw.reshape(1, _D), lb.reshape(1, _D))


def _cnorm_body(x_ref, o_ref):
    o_ref[...] = _nrm(x_ref[...])


def _cnorm_call(ck):
    blk = 2000
    return pl.pallas_call(
        _cnorm_body,
        grid=(_C // blk,),
        in_specs=[pl.BlockSpec((blk, _QD), lambda c: (c, 0))],
        out_specs=pl.BlockSpec((blk, _QD), lambda c: (c, 0)),
        out_shape=jax.ShapeDtypeStruct((_C, _QD), jnp.float32),
    )(ck)


@functools.cache
def _make_sc_gather():
    mesh = plsc.VectorSubcoreMesh(core_axis_name="c", subcore_axis_name="s")

    @functools.partial(
        pl.kernel, mesh=mesh,
        out_type=jax.ShapeDtypeStruct((_NROWS, _D), jnp.float32),
        scratch_types=[
            pltpu.VMEM((_RPW,), jnp.int32),
            pltpu.VMEM((_RPW, _D), jnp.float32),
            pltpu.SemaphoreType.DMA,
        ],
    )
    def sc_gather(table_hbm, idx_hbm, out_hbm, idx_v, rows_v, sem):
        wid = lax.axis_index("s") * _SC_NC + lax.axis_index("c")
        base = wid * _RPW
        pltpu.sync_copy(idx_hbm.at[pl.ds(base, _RPW)], idx_v)
        pltpu.async_copy(table_hbm.at[idx_v], rows_v, sem).wait()
        pltpu.sync_copy(rows_v, out_hbm.at[pl.ds(base, _RPW)])

    return sc_gather


def _sc_gather(table, idx):
    return _make_sc_gather()(table, idx)


def kernel(input_repr, W_in1, b_in1, ln_in_w, ln_in_b, Wp1, bp1, Wp2, bp2,
           lnq_w, lnq_b, Wg, bg, attn_in_w, attn_in_b, attn_out_w,
           attn_out_b, Wt1, bt1, lnt_w, lnt_b, Wt2, bt2, Wc1, bc1, Wc2,
           bc2, Wa1, ba1, Wa2, ba2, corpus_keys, corpus_values):
    Wq = attn_in_w[:_D]
    Wk = attn_in_w[_D:2 * _D]
    Wv = attn_in_w[2 * _D:]
    bq = attn_in_b[:_D]
    bk = attn_in_b[_D:2 * _D]
    bv = attn_in_b[2 * _D:]

    current = _input_call(input_repr, W_in1, b_in1, ln_in_w, ln_in_b)
    corpus_norm = _cnorm_call(corpus_keys)

    tw = (Wq, bq, Wk, bk, Wv, bv, attn_out_w, attn_out_b, Wt1, bt1,
          lnt_w, lnt_b, Wt2, bt2)
    qw = (Wp1, bp1, Wp2, bp2, lnq_w, lnq_b, Wg, bg)

    idx = _stepA_call(current, corpus_norm, Wp1, bp1, Wp2, bp2,
                      lnq_w, lnq_b)
    docs = _sc_gather(corpus_values, idx.reshape(_NROWS))
    docs = docs.reshape(_B, _K * _D)
    ctx = current
    for step in range(1, _STEPS):
        idx, current, ctx = _stepB_call(step == 1, current, ctx, docs,
                                        corpus_norm, *tw, *qw)
        docs = _sc_gather(corpus_values, idx.reshape(_NROWS))
        docs = docs.reshape(_B, _K * _D)
    return _final_call(current, ctx, docs, *tw, Wa1, ba1, Wa2, ba2)
